# baseline (speedup 1.0000x reference)
import jax
import jax.numpy as jnp
from jax.experimental import pallas as pl
from jax.experimental.pallas import tpu as pltpu

_FBLK = 16
_NBUF = 4


def _pe_add_kernel(inp_hbm, pos_ref, out_hbm, in_v, out_v, pos_t,
                   in_sem, out_sem):
    nchunks = pl.num_programs(0)
    gpb = nchunks // inp_hbm.shape[0]
    i = pl.program_id(0)
    slot = jax.lax.rem(i, _NBUF)

    def in_copy(c, s):
        b = c // gpb
        f0 = jax.lax.rem(c, gpb) * _FBLK
        return pltpu.make_async_copy(
            inp_hbm.at[b, pl.ds(f0, _FBLK)], in_v.at[s], in_sem.at[s])

    def out_copy(c, s):
        b = c // gpb
        f0 = jax.lax.rem(c, gpb) * _FBLK
        return pltpu.make_async_copy(
            out_v.at[s], out_hbm.at[b, pl.ds(f0, _FBLK)], out_sem.at[s])

    @pl.when(i == 0)
    def _():
        pos_t[...] = pos_ref[...].T
        for k in range(_NBUF - 1):
            in_copy(k, k).start()

    @pl.when(i + _NBUF - 1 < nchunks)
    def _():
        in_copy(i + _NBUF - 1, jax.lax.rem(i + _NBUF - 1, _NBUF)).start()

    in_copy(i, slot).wait()

    @pl.when(i >= _NBUF)
    def _():
        out_copy(i - _NBUF, slot).wait()

    f0 = jax.lax.rem(i, gpb) * _FBLK
    slab = pos_t[pl.ds(f0, _FBLK), :]
    out_v[slot] = in_v[slot] + slab[:, None, :]
    out_copy(i, slot).start()

    @pl.when(i == nchunks - 1)
    def _():
        for k in range(_NBUF):
            c = nchunks - _NBUF + k
            out_copy(c, jax.lax.rem(c, _NBUF)).wait()


def kernel(input_tensor, pos_table):
    batch, feature, height, seq = input_tensor.shape
    nchunks = batch * (feature // _FBLK)

    return pl.pallas_call(
        _pe_add_kernel,
        grid=(nchunks,),
        in_specs=[
            pl.BlockSpec(memory_space=pl.ANY),
            pl.BlockSpec((seq, feature), lambda i: (0, 0)),
        ],
        out_specs=pl.BlockSpec(memory_space=pl.ANY),
        out_shape=jax.ShapeDtypeStruct(input_tensor.shape, input_tensor.dtype),
        scratch_shapes=[
            pltpu.VMEM((_NBUF, _FBLK, height, seq), input_tensor.dtype),
            pltpu.VMEM((_NBUF, _FBLK, height, seq), input_tensor.dtype),
            pltpu.VMEM((feature, seq), input_tensor.dtype),
            pltpu.SemaphoreType.DMA((_NBUF,)),
            pltpu.SemaphoreType.DMA((_NBUF,)),
        ],
    )(input_tensor, pos_table)
